# baseline (speedup 1.0000x reference)
import functools
import math

import jax
import jax.numpy as jnp
import numpy as np
from jax import lax
from jax.experimental import pallas as pl
from jax.experimental.pallas import tpu as pltpu

_SLOPE = 0.2
_VMEM = 56 * 1024 * 1024
_BF = jnp.bfloat16


def _leaky(x):
    return jnp.where(x > 0, x, _SLOPE * x)


def _hpad(h):
    H, W, C = h.shape
    zr = jnp.zeros((1, W, C), h.dtype)
    h = jnp.concatenate([zr, h, zr], 0)
    zc = jnp.zeros((H + 2, 1, C), h.dtype)
    return jnp.concatenate([zc, h, zc], 1)


def _conv_s1(h, w9, b, relu=True):
    H, W, C = h.shape
    Cout = w9.shape[-1]
    hp = _hpad(h)
    wr = w9.reshape(9 * C, Cout)
    rc = max(8, (8 << 20) // max(1, W * 9 * C * 2))
    rc = min(rc, H)
    outs = []
    for r0 in range(0, H, rc):
        r = min(rc, H - r0)
        taps = [hp[r0 + dy:r0 + dy + r, dx:dx + W, :]
                for dy in range(3) for dx in range(3)]
        patch = jnp.concatenate(taps, -1).reshape(r * W, 9 * C)
        outs.append(jnp.dot(patch, wr, preferred_element_type=jnp.float32))
    acc = jnp.concatenate(outs, 0) + b
    if relu:
        acc = _leaky(acc)
    return acc.reshape(H, W, Cout).astype(_BF)


def _fea_tiled_kernel(*refs, ho, wo, cin, th):
    xm_ref = refs[0]
    halo = [refs[1 + k] for k in range(6)]
    w1_ref, b1_ref, w2_ref, b2_ref, o_ref = refs[7:]
    cmid = w1_ref.shape[-1]
    x = jnp.concatenate([xm_ref[0]] + [h[0] for h in halo], 0)
    xq = x.reshape(th + 3, 2, wo + 1, 2 * cin)
    rows1 = th + 2
    taps = []
    for dy in range(3):
        s = xq[dy // 2:dy // 2 + rows1, dy % 2]
        taps.append(s[:, :wo, :cin])
        taps.append(s[:, :wo, cin:])
        taps.append(s[:, 1:, :cin])
    patch = jnp.concatenate(taps, -1).reshape(rows1 * wo, 9 * cin)
    h = jnp.dot(patch, w1_ref[...], preferred_element_type=jnp.float32)
    h = _leaky(h + b1_ref[...]).reshape(rows1, wo, cmid)
    g = lax.broadcasted_iota(jnp.int32, (rows1, 1, 1), 0) \
        + th * pl.program_id(1) - 1
    h = jnp.where((g >= 0) & (g < ho), h, 0.0).astype(_BF)
    zc = jnp.zeros((rows1, 1, cmid), _BF)
    hc = jnp.concatenate([zc, h, zc], 1)
    wr = w2_ref[...].reshape(9 * cmid, cmid)
    taps2 = [hc[dy:dy + th, dx:dx + wo, :]
             for dy in range(3) for dx in range(3)]
    p2 = jnp.concatenate(taps2, -1).reshape(th * wo, 9 * cmid)
    acc = jnp.dot(p2, wr, preferred_element_type=jnp.float32) + b2_ref[...]
    o_ref[0] = _leaky(acc).reshape(th, wo, cmid).astype(_BF)


def _fea_level_tiled(h, w1, b1, w2, b2, th):
    N, H, W, Cin = h.shape
    Cmid = w1.shape[-1]
    Ho, Wo = H // 2, W // 2
    nrt = Ho // th
    hp = jnp.pad(h, ((0, 0), (3, 3), (1, 1), (0, 0)))
    hf = hp.reshape(N, H + 6, Wo + 1, 2 * Cin)
    w1r = w1.reshape(9 * Cin, Cmid).astype(_BF)
    w2r = w2.reshape(9, Cmid, Cmid).astype(_BF)
    b1r = b1.reshape(1, Cmid).astype(jnp.float32)
    b2r = b2.reshape(1, Cmid).astype(jnp.float32)
    row_spec = pl.BlockSpec((1, 2 * th, Wo + 1, 2 * Cin),
                            lambda n, r: (n, r, 0, 0))
    halo_specs = [
        pl.BlockSpec((1, 1, Wo + 1, 2 * Cin),
                     functools.partial(
                         lambda k, n, r: (n, 2 * th * (r + 1) + k, 0, 0), k))
        for k in range(6)
    ]
    return pl.pallas_call(
        functools.partial(_fea_tiled_kernel, ho=Ho, wo=Wo, cin=Cin, th=th),
        out_shape=jax.ShapeDtypeStruct((N, Ho, Wo, Cmid), _BF),
        grid=(N, nrt),
        in_specs=[row_spec] + halo_specs + [
            pl.BlockSpec((9 * Cin, Cmid), lambda n, r: (0, 0)),
            pl.BlockSpec((1, Cmid), lambda n, r: (0, 0)),
            pl.BlockSpec((9, Cmid, Cmid), lambda n, r: (0, 0, 0)),
            pl.BlockSpec((1, Cmid), lambda n, r: (0, 0)),
        ],
        out_specs=pl.BlockSpec((1, th, Wo, Cmid), lambda n, r: (n, r, 0, 0)),
        compiler_params=pltpu.CompilerParams(
            dimension_semantics=("parallel", "parallel"),
            vmem_limit_bytes=_VMEM,
        ),
    )(hf, hf, hf, hf, hf, hf, hf, w1r, b1r, w2r, b2r)


def _fea_folded_kernel(x_ref, w1_ref, b1_ref, w2_ref, b2_ref, o_ref, *,
                       ho, nb, ko, cm):
    x = x_ref[0]
    k1 = x.shape[-1]
    xq = x.reshape(ho + 1, 2, nb, k1)
    m = ho * nb
    acc = jnp.broadcast_to(b1_ref[...], (m, ko * cm)).astype(jnp.float32)
    taps = (xq[:ho, 0], xq[:ho, 1], xq[1:ho + 1, 0])
    for dy in range(3):
        acc = acc + jnp.dot(taps[dy].reshape(m, k1), w1_ref[dy],
                            preferred_element_type=jnp.float32)
    h1 = _leaky(acc).astype(_BF).reshape(ho, nb, ko * cm)
    z = jnp.zeros((ho, 1, cm), _BF)
    s1 = jnp.concatenate([z, h1[:, :-1, -cm:]], 1)
    s2 = jnp.concatenate([h1[:, 1:, :cm], z], 1)
    p2 = jnp.concatenate([s1, h1, s2], -1)
    k2 = p2.shape[-1]
    zr = jnp.zeros((1, nb, k2), _BF)
    p2 = jnp.concatenate([zr, p2, zr], 0)
    acc2 = jnp.broadcast_to(b2_ref[...], (m, ko * cm)).astype(jnp.float32)
    for dy in range(3):
        acc2 = acc2 + jnp.dot(p2[dy:dy + ho].reshape(m, k2), w2_ref[dy],
                              preferred_element_type=jnp.float32)
    o_ref[0] = _leaky(acc2).astype(_BF).reshape(ho, nb, ko * cm)


def _fold_w(w, s, ko):
    cin, cout = w.shape[2], w.shape[3]
    J = s * ko + 2
    mats = []
    for dy in range(3):
        acc = 0
        for dx in range(3):
            e = np.zeros((J, ko), np.float32)
            for xo in range(ko):
                e[s * xo + dx, xo] = 1.0
            acc = acc + jnp.kron(jnp.asarray(e), w[dy, dx])
        mats.append(acc)
    return jnp.stack(mats, 0).astype(_BF)


def _fea_level_folded(h, w1, b1, w2, b2, ko):
    N, H, W, Cin = h.shape
    Cm = w1.shape[-1]
    Ho, Wo = H // 2, W // 2
    NB = Wo // ko
    sw = 2 * ko
    xp = jnp.pad(h, ((0, 0), (1, 1), (1, 1), (0, 0)))
    a = xp[:, :, :W, :].reshape(N, H + 2, NB, sw * Cin)
    nxt = jnp.pad(xp[:, :, sw:, :], ((0, 0), (0, 0), (0, sw - 2), (0, 0)))
    nxt = nxt.reshape(N, H + 2, NB, sw * Cin)[..., :2 * Cin]
    xpre = jnp.concatenate([a, nxt], -1)
    k1 = (sw + 2) * Cin
    k2 = (ko + 2) * Cm
    w1f = _fold_w(w1, 2, ko)
    w2f = _fold_w(w2, 1, ko)
    b1f = jnp.tile(b1.reshape(1, Cm), (1, ko)).astype(jnp.float32)
    b2f = jnp.tile(b2.reshape(1, Cm), (1, ko)).astype(jnp.float32)
    out = pl.pallas_call(
        functools.partial(_fea_folded_kernel, ho=Ho, nb=NB, ko=ko, cm=Cm),
        out_shape=jax.ShapeDtypeStruct((N, Ho, NB, ko * Cm), _BF),
        grid=(N,),
        in_specs=[
            pl.BlockSpec((1, H + 2, NB, k1), lambda n: (n, 0, 0, 0)),
            pl.BlockSpec((3, k1, ko * Cm), lambda n: (0, 0, 0)),
            pl.BlockSpec((1, ko * Cm), lambda n: (0, 0)),
            pl.BlockSpec((3, k2, ko * Cm), lambda n: (0, 0, 0)),
            pl.BlockSpec((1, ko * Cm), lambda n: (0, 0)),
        ],
        out_specs=pl.BlockSpec((1, Ho, NB, ko * Cm), lambda n: (n, 0, 0, 0)),
        compiler_params=pltpu.CompilerParams(
            dimension_semantics=("parallel",),
            vmem_limit_bytes=_VMEM,
        ),
    )(xpre, w1f, b1f, w2f, b2f)
    return out.reshape(N, Ho, Wo, Cm)


def _fea_kernel(xf_ref, w1_ref, b1_ref, w2_ref, b2_ref, o_ref, *, ho, wo, cin):
    xf = xf_ref[0]
    cmid = w1_ref.shape[-1]
    xq = xf.reshape(ho + 1, 2, wo + 1, 2 * cin)
    s0 = xq[:ho, 0]
    s1 = xq[:ho, 1]
    s2 = xq[1:ho + 1, 0]
    taps = []
    for s in (s0, s1, s2):
        taps.append(s[:, :wo, :cin])
        taps.append(s[:, :wo, cin:])
        taps.append(s[:, 1:, :cin])
    patch = jnp.concatenate(taps, -1).reshape(ho * wo, 9 * cin)
    h = jnp.dot(patch, w1_ref[...], preferred_element_type=jnp.float32)
    h = _leaky(h + b1_ref[...]).reshape(ho, wo, cmid).astype(_BF)
    o_ref[0] = _conv_s1(h, w2_ref[...], b2_ref[...], relu=True)


def _fea_level(h, w1, b1, w2, b2):
    N, H, W, Cin = h.shape
    Cmid = w1.shape[-1]
    Ho, Wo = H // 2, W // 2
    hp = jnp.pad(h, ((0, 0), (1, 1), (1, 1), (0, 0)))
    hf = hp.reshape(N, H + 2, Wo + 1, 2 * Cin)
    w1r = w1.reshape(9 * Cin, Cmid).astype(_BF)
    w2r = w2.reshape(9, Cmid, Cmid).astype(_BF)
    b1r = b1.reshape(1, Cmid).astype(jnp.float32)
    b2r = b2.reshape(1, Cmid).astype(jnp.float32)
    return pl.pallas_call(
        functools.partial(_fea_kernel, ho=Ho, wo=Wo, cin=Cin),
        out_shape=jax.ShapeDtypeStruct((N, Ho, Wo, Cmid), _BF),
        grid=(N,),
        in_specs=[
            pl.BlockSpec((1, H + 2, Wo + 1, 2 * Cin), lambda n: (n, 0, 0, 0)),
            pl.BlockSpec((9 * Cin, Cmid), lambda n: (0, 0)),
            pl.BlockSpec((1, Cmid), lambda n: (0, 0)),
            pl.BlockSpec((9, Cmid, Cmid), lambda n: (0, 0, 0)),
            pl.BlockSpec((1, Cmid), lambda n: (0, 0)),
        ],
        out_specs=pl.BlockSpec((1, Ho, Wo, Cmid), lambda n: (n, 0, 0, 0)),
        compiler_params=pltpu.CompilerParams(
            dimension_semantics=("parallel",),
            vmem_limit_bytes=_VMEM,
        ),
    )(hf, w1r, b1r, w2r, b2r)


def _corr_ext(f2, f3, xs, cf):
    H, W, _ = f2.shape
    scale = 1.0 / math.sqrt(cf)
    posx = (lax.broadcasted_iota(jnp.int32, (1, 1, W + 4), 2) - 2
            ).astype(jnp.float32)
    rc = min(H, max(8, (4 << 20) // max(1, W * W * 4)))
    rows = []
    for r0 in range(0, H, rc):
        r = min(rc, H - r0)
        corr = lax.dot_general(
            f2[r0:r0 + r], f3[r0:r0 + r],
            (((2,), (2,)), ((0,), (0,))),
            preferred_element_type=jnp.float32) * scale
        corr_x = jnp.pad(corr, ((0, 0), (0, 0), (4, 4)))
        hat0 = jnp.maximum(
            0.0, 1.0 - jnp.abs(xs[r0:r0 + r, :, None] - posx))
        js = []
        for j in (-2, -1, 0, 1, 2):
            sl = corr_x[:, :, 2 + j:2 + j + W + 4]
            js.append(jnp.sum(sl * hat0, -1, keepdims=True))
        rows.append(jnp.concatenate(js, -1))
    return jnp.concatenate(rows, 0)


def _head(h, wv_ref, b_ref):
    H, W, C = h.shape
    hp = _hpad(h)
    wv = wv_ref[...].astype(jnp.float32)
    rc = min(H, max(8, (4 << 20) // max(1, W * 9 * C * 4)))
    outs = []
    for r0 in range(0, H, rc):
        r = min(rc, H - r0)
        taps = [hp[r0 + dy:r0 + dy + r, dx:dx + W, :]
                for dy in range(3) for dx in range(3)]
        patch = jnp.concatenate(taps, -1).astype(jnp.float32)
        outs.append(jnp.sum(patch * wv, -1))
    return jnp.concatenate(outs, 0) + b_ref[0, 0]


def _dec_kernel(*refs, cf, scale, has_disp):
    if has_disp:
        f2_ref, f3_ref, dp_ref = refs[:3]
        wrefs = refs[3:-1]
    else:
        f2_ref, f3_ref = refs[:2]
        wrefs = refs[2:-1]
    o_ref = refs[-1]
    f2 = f2_ref[0]
    f3 = f3_ref[0]
    ho, w, _ = f2.shape
    wiota = lax.broadcasted_iota(jnp.int32, (ho, w), 1).astype(jnp.float32)
    if has_disp:
        du = jnp.repeat(jnp.repeat(dp_ref[0], 2, 0), 2, 1) * scale
        xs = du + wiota
    else:
        du = None
        xs = wiota
    ext = _corr_ext(f2, f3, xs, cf)
    if has_disp:
        ext = jnp.concatenate([ext, du[:, :, None]], -1)
    h = jnp.concatenate([f2, ext.astype(_BF)], -1)
    for i in range(5):
        h = _conv_s1(h, wrefs[2 * i][...], wrefs[2 * i + 1][...], relu=True)
    o_ref[0] = _head(h, wrefs[10], wrefs[11])


def _dec_level(F, disp_prev, layers, cf, scale):
    N8, Ho, W, _ = F.shape
    N = N8 // 2
    has_disp = disp_prev is not None
    ops = []
    in_specs = [
        pl.BlockSpec((1, Ho, W, cf), lambda n: (n, 0, 0, 0)),
        pl.BlockSpec((1, Ho, W, cf), lambda n: (n + N, 0, 0, 0)),
    ]
    if has_disp:
        Hp, Wp = disp_prev.shape[1:3]
        ops.append(disp_prev)
        in_specs.append(pl.BlockSpec((1, Hp, Wp), lambda n: (n, 0, 0)))
    for i, (wl, bl) in enumerate(layers):
        cin, cout = wl.shape[2], wl.shape[3]
        if i == 5:
            ops.append(wl.reshape(1, 1, 9 * cin).astype(_BF))
            in_specs.append(pl.BlockSpec((1, 1, 9 * cin), lambda n: (0, 0, 0)))
            ops.append(bl.reshape(1, 1).astype(jnp.float32))
            in_specs.append(pl.BlockSpec((1, 1), lambda n: (0, 0)))
        else:
            ops.append(wl.reshape(9, cin, cout).astype(_BF))
            in_specs.append(pl.BlockSpec((9, cin, cout), lambda n: (0, 0, 0)))
            ops.append(bl.reshape(1, cout).astype(jnp.float32))
            in_specs.append(pl.BlockSpec((1, cout), lambda n: (0, 0)))
    return pl.pallas_call(
        functools.partial(_dec_kernel, cf=cf, scale=scale, has_disp=has_disp),
        out_shape=jax.ShapeDtypeStruct((N, Ho, W), jnp.float32),
        grid=(N,),
        in_specs=in_specs,
        out_specs=pl.BlockSpec((1, Ho, W), lambda n: (n, 0, 0)),
        compiler_params=pltpu.CompilerParams(
            dimension_semantics=("parallel",),
            vmem_limit_bytes=_VMEM,
        ),
    )(F, F, *ops)


def kernel(image2, image3, fea_0_c1_w, fea_0_c1_b, fea_0_c2_w, fea_0_c2_b, fea_1_c1_w, fea_1_c1_b, fea_1_c2_w, fea_1_c2_b, fea_2_c1_w, fea_2_c1_b, fea_2_c2_w, fea_2_c2_b, fea_3_c1_w, fea_3_c1_b, fea_3_c2_w, fea_3_c2_b, fea_4_c1_w, fea_4_c1_b, fea_4_c2_w, fea_4_c2_b, fea_5_c1_w, fea_5_c1_b, fea_5_c2_w, fea_5_c2_b, dec_6_0_w, dec_6_0_b, dec_6_1_w, dec_6_1_b, dec_6_2_w, dec_6_2_b, dec_6_3_w, dec_6_3_b, dec_6_4_w, dec_6_4_b, dec_6_5_w, dec_6_5_b, dec_5_0_w, dec_5_0_b, dec_5_1_w, dec_5_1_b, dec_5_2_w, dec_5_2_b, dec_5_3_w, dec_5_3_b, dec_5_4_w, dec_5_4_b, dec_5_5_w, dec_5_5_b, dec_4_0_w, dec_4_0_b, dec_4_1_w, dec_4_1_b, dec_4_2_w, dec_4_2_b, dec_4_3_w, dec_4_3_b, dec_4_4_w, dec_4_4_b, dec_4_5_w, dec_4_5_b, dec_3_0_w, dec_3_0_b, dec_3_1_w, dec_3_1_b, dec_3_2_w, dec_3_2_b, dec_3_3_w, dec_3_3_b, dec_3_4_w, dec_3_4_b, dec_3_5_w, dec_3_5_b, dec_2_0_w, dec_2_0_b, dec_2_1_w, dec_2_1_b, dec_2_2_w, dec_2_2_b, dec_2_3_w, dec_2_3_b, dec_2_4_w, dec_2_4_b, dec_2_5_w, dec_2_5_b):
    args = locals()
    p = [args[n] for n in _ARG_ORDER]
    return _kernel_impl(image2, image3, *p)


_ARG_ORDER = []
for _l in range(6):
    for _c in ("c1", "c2"):
        _ARG_ORDER += [f"fea_{_l}_{_c}_w", f"fea_{_l}_{_c}_b"]
for _l in (6, 5, 4, 3, 2):
    for _i in range(6):
        _ARG_ORDER += [f"dec_{_l}_{_i}_w", f"dec_{_l}_{_i}_b"]


def _kernel_impl(image2, image3, *p):
    names = []
    for lvl in range(6):
        for c in ("c1", "c2"):
            names += [f"fea_{lvl}_{c}_w", f"fea_{lvl}_{c}_b"]
    for lvl in (6, 5, 4, 3, 2):
        for i in range(6):
            names += [f"dec_{lvl}_{i}_w", f"dec_{lvl}_{i}_b"]
    prm = dict(zip(names, p))

    x2 = jnp.transpose(image2, (0, 2, 3, 1)).astype(_BF)
    x3 = jnp.transpose(image3, (0, 2, 3, 1)).astype(_BF)
    h = jnp.concatenate([x2, x3], 0)

    feats = []
    tile_rows = {0: 32, 1: 16}
    for lvl in range(6):
        wb = (prm[f"fea_{lvl}_c1_w"], prm[f"fea_{lvl}_c1_b"],
              prm[f"fea_{lvl}_c2_w"], prm[f"fea_{lvl}_c2_b"])
        if lvl in tile_rows:
            h = _fea_level_tiled(h, *wb, tile_rows[lvl])
        else:
            h = _fea_level(h, *wb)
        feats.append(h)

    cfs = {2: 32, 3: 64, 4: 96, 5: 128, 6: 192}
    disp = None
    disps = {}
    for lvl in (6, 5, 4, 3, 2):
        layers = [(prm[f"dec_{lvl}_{i}_w"], prm[f"dec_{lvl}_{i}_b"])
                  for i in range(6)]
        disp = _dec_level(feats[lvl - 1], disp, layers, cfs[lvl],
                          20.0 / (2 ** lvl))
        disps[lvl] = disp

    out = tuple(disps[lvl][:, None, :, :] for lvl in (2, 3, 4, 5, 6))
    return out


# speedup vs baseline: 1.0331x; 1.0331x over previous
import functools
import math

import jax
import jax.numpy as jnp
import numpy as np
from jax import lax
from jax.experimental import pallas as pl
from jax.experimental.pallas import tpu as pltpu

_SLOPE = 0.2
_VMEM = 56 * 1024 * 1024
_BF = jnp.bfloat16


def _leaky(x):
    return jnp.where(x > 0, x, _SLOPE * x)


def _hpad(h):
    H, W, C = h.shape
    zr = jnp.zeros((1, W, C), h.dtype)
    h = jnp.concatenate([zr, h, zr], 0)
    zc = jnp.zeros((H + 2, 1, C), h.dtype)
    return jnp.concatenate([zc, h, zc], 1)


def _conv_s1(h, w9, b, relu=True):
    H, W, C = h.shape
    Cout = w9.shape[-1]
    hp = _hpad(h)
    wr = w9.reshape(9 * C, Cout)
    rc = max(8, (4 << 20) // max(1, W * 9 * C * 2))
    rc = min(rc, H)
    outs = []
    for r0 in range(0, H, rc):
        r = min(rc, H - r0)
        taps = [hp[r0 + dy:r0 + dy + r, dx:dx + W, :]
                for dy in range(3) for dx in range(3)]
        patch = jnp.concatenate(taps, -1).reshape(r * W, 9 * C)
        outs.append(jnp.dot(patch, wr, preferred_element_type=jnp.float32))
    acc = jnp.concatenate(outs, 0) + b
    if relu:
        acc = _leaky(acc)
    return acc.reshape(H, W, Cout).astype(_BF)


def _fea_tiled_kernel(*refs, ho, wo, cin, th):
    xm_ref = refs[0]
    halo = [refs[1 + k] for k in range(6)]
    w1_ref, b1_ref, w2_ref, b2_ref, o_ref = refs[7:]
    cmid = w1_ref.shape[-1]
    x = jnp.concatenate([xm_ref[0]] + [h[0] for h in halo], 0)
    xq = x.reshape(th + 3, 2, wo + 1, 2 * cin)
    rows1 = th + 2
    taps = []
    for dy in range(3):
        s = xq[dy // 2:dy // 2 + rows1, dy % 2]
        taps.append(s[:, :wo, :cin])
        taps.append(s[:, :wo, cin:])
        taps.append(s[:, 1:, :cin])
    patch = jnp.concatenate(taps, -1).reshape(rows1 * wo, 9 * cin)
    h = jnp.dot(patch, w1_ref[...], preferred_element_type=jnp.float32)
    h = _leaky(h + b1_ref[...]).reshape(rows1, wo, cmid)
    g = lax.broadcasted_iota(jnp.int32, (rows1, 1, 1), 0) \
        + th * pl.program_id(1) - 1
    h = jnp.where((g >= 0) & (g < ho), h, 0.0).astype(_BF)
    zc = jnp.zeros((rows1, 1, cmid), _BF)
    hc = jnp.concatenate([zc, h, zc], 1)
    wr = w2_ref[...].reshape(9 * cmid, cmid)
    taps2 = [hc[dy:dy + th, dx:dx + wo, :]
             for dy in range(3) for dx in range(3)]
    p2 = jnp.concatenate(taps2, -1).reshape(th * wo, 9 * cmid)
    acc = jnp.dot(p2, wr, preferred_element_type=jnp.float32) + b2_ref[...]
    o_ref[0] = _leaky(acc).reshape(th, wo, cmid).astype(_BF)


def _fea_level_tiled(h, w1, b1, w2, b2, th):
    N, H, W, Cin = h.shape
    Cmid = w1.shape[-1]
    Ho, Wo = H // 2, W // 2
    nrt = Ho // th
    hp = jnp.pad(h, ((0, 0), (3, 3), (1, 1), (0, 0)))
    hf = hp.reshape(N, H + 6, Wo + 1, 2 * Cin)
    w1r = w1.reshape(9 * Cin, Cmid).astype(_BF)
    w2r = w2.reshape(9, Cmid, Cmid).astype(_BF)
    b1r = b1.reshape(1, Cmid).astype(jnp.float32)
    b2r = b2.reshape(1, Cmid).astype(jnp.float32)
    row_spec = pl.BlockSpec((1, 2 * th, Wo + 1, 2 * Cin),
                            lambda n, r: (n, r, 0, 0))
    halo_specs = [
        pl.BlockSpec((1, 1, Wo + 1, 2 * Cin),
                     functools.partial(
                         lambda k, n, r: (n, 2 * th * (r + 1) + k, 0, 0), k))
        for k in range(6)
    ]
    return pl.pallas_call(
        functools.partial(_fea_tiled_kernel, ho=Ho, wo=Wo, cin=Cin, th=th),
        out_shape=jax.ShapeDtypeStruct((N, Ho, Wo, Cmid), _BF),
        grid=(N, nrt),
        in_specs=[row_spec] + halo_specs + [
            pl.BlockSpec((9 * Cin, Cmid), lambda n, r: (0, 0)),
            pl.BlockSpec((1, Cmid), lambda n, r: (0, 0)),
            pl.BlockSpec((9, Cmid, Cmid), lambda n, r: (0, 0, 0)),
            pl.BlockSpec((1, Cmid), lambda n, r: (0, 0)),
        ],
        out_specs=pl.BlockSpec((1, th, Wo, Cmid), lambda n, r: (n, r, 0, 0)),
        compiler_params=pltpu.CompilerParams(
            dimension_semantics=("parallel", "parallel"),
            vmem_limit_bytes=_VMEM,
        ),
    )(hf, hf, hf, hf, hf, hf, hf, w1r, b1r, w2r, b2r)


def _fea_folded_kernel(x_ref, w1_ref, b1_ref, w2_ref, b2_ref, o_ref, *,
                       ho, nb, ko, cm):
    x = x_ref[0]
    k1 = x.shape[-1]
    xq = x.reshape(ho + 1, 2, nb, k1)
    m = ho * nb
    acc = jnp.broadcast_to(b1_ref[...], (m, ko * cm)).astype(jnp.float32)
    taps = (xq[:ho, 0], xq[:ho, 1], xq[1:ho + 1, 0])
    for dy in range(3):
        acc = acc + jnp.dot(taps[dy].reshape(m, k1), w1_ref[dy],
                            preferred_element_type=jnp.float32)
    h1 = _leaky(acc).astype(_BF).reshape(ho, nb, ko * cm)
    z = jnp.zeros((ho, 1, cm), _BF)
    s1 = jnp.concatenate([z, h1[:, :-1, -cm:]], 1)
    s2 = jnp.concatenate([h1[:, 1:, :cm], z], 1)
    p2 = jnp.concatenate([s1, h1, s2], -1)
    k2 = p2.shape[-1]
    zr = jnp.zeros((1, nb, k2), _BF)
    p2 = jnp.concatenate([zr, p2, zr], 0)
    acc2 = jnp.broadcast_to(b2_ref[...], (m, ko * cm)).astype(jnp.float32)
    for dy in range(3):
        acc2 = acc2 + jnp.dot(p2[dy:dy + ho].reshape(m, k2), w2_ref[dy],
                              preferred_element_type=jnp.float32)
    o_ref[0] = _leaky(acc2).astype(_BF).reshape(ho, nb, ko * cm)


def _fold_w(w, s, ko):
    cin, cout = w.shape[2], w.shape[3]
    J = s * ko + 2
    mats = []
    for dy in range(3):
        acc = 0
        for dx in range(3):
            e = np.zeros((J, ko), np.float32)
            for xo in range(ko):
                e[s * xo + dx, xo] = 1.0
            acc = acc + jnp.kron(jnp.asarray(e), w[dy, dx])
        mats.append(acc)
    return jnp.stack(mats, 0).astype(_BF)


def _fea_level_folded(h, w1, b1, w2, b2, ko):
    N, H, W, Cin = h.shape
    Cm = w1.shape[-1]
    Ho, Wo = H // 2, W // 2
    NB = Wo // ko
    sw = 2 * ko
    xp = jnp.pad(h, ((0, 0), (1, 1), (1, 1), (0, 0)))
    a = xp[:, :, :W, :].reshape(N, H + 2, NB, sw * Cin)
    nxt = jnp.pad(xp[:, :, sw:, :], ((0, 0), (0, 0), (0, sw - 2), (0, 0)))
    nxt = nxt.reshape(N, H + 2, NB, sw * Cin)[..., :2 * Cin]
    xpre = jnp.concatenate([a, nxt], -1)
    k1 = (sw + 2) * Cin
    k2 = (ko + 2) * Cm
    w1f = _fold_w(w1, 2, ko)
    w2f = _fold_w(w2, 1, ko)
    b1f = jnp.tile(b1.reshape(1, Cm), (1, ko)).astype(jnp.float32)
    b2f = jnp.tile(b2.reshape(1, Cm), (1, ko)).astype(jnp.float32)
    out = pl.pallas_call(
        functools.partial(_fea_folded_kernel, ho=Ho, nb=NB, ko=ko, cm=Cm),
        out_shape=jax.ShapeDtypeStruct((N, Ho, NB, ko * Cm), _BF),
        grid=(N,),
        in_specs=[
            pl.BlockSpec((1, H + 2, NB, k1), lambda n: (n, 0, 0, 0)),
            pl.BlockSpec((3, k1, ko * Cm), lambda n: (0, 0, 0)),
            pl.BlockSpec((1, ko * Cm), lambda n: (0, 0)),
            pl.BlockSpec((3, k2, ko * Cm), lambda n: (0, 0, 0)),
            pl.BlockSpec((1, ko * Cm), lambda n: (0, 0)),
        ],
        out_specs=pl.BlockSpec((1, Ho, NB, ko * Cm), lambda n: (n, 0, 0, 0)),
        compiler_params=pltpu.CompilerParams(
            dimension_semantics=("parallel",),
            vmem_limit_bytes=_VMEM,
        ),
    )(xpre, w1f, b1f, w2f, b2f)
    return out.reshape(N, Ho, Wo, Cm)


def _fea_kernel(xf_ref, w1_ref, b1_ref, w2_ref, b2_ref, o_ref, *, ho, wo, cin):
    xf = xf_ref[0]
    cmid = w1_ref.shape[-1]
    xq = xf.reshape(ho + 1, 2, wo + 1, 2 * cin)
    s0 = xq[:ho, 0]
    s1 = xq[:ho, 1]
    s2 = xq[1:ho + 1, 0]
    taps = []
    for s in (s0, s1, s2):
        taps.append(s[:, :wo, :cin])
        taps.append(s[:, :wo, cin:])
        taps.append(s[:, 1:, :cin])
    patch = jnp.concatenate(taps, -1).reshape(ho * wo, 9 * cin)
    h = jnp.dot(patch, w1_ref[...], preferred_element_type=jnp.float32)
    h = _leaky(h + b1_ref[...]).reshape(ho, wo, cmid).astype(_BF)
    o_ref[0] = _conv_s1(h, w2_ref[...], b2_ref[...], relu=True)


def _fea_level(h, w1, b1, w2, b2):
    N, H, W, Cin = h.shape
    Cmid = w1.shape[-1]
    Ho, Wo = H // 2, W // 2
    hp = jnp.pad(h, ((0, 0), (1, 1), (1, 1), (0, 0)))
    hf = hp.reshape(N, H + 2, Wo + 1, 2 * Cin)
    w1r = w1.reshape(9 * Cin, Cmid).astype(_BF)
    w2r = w2.reshape(9, Cmid, Cmid).astype(_BF)
    b1r = b1.reshape(1, Cmid).astype(jnp.float32)
    b2r = b2.reshape(1, Cmid).astype(jnp.float32)
    return pl.pallas_call(
        functools.partial(_fea_kernel, ho=Ho, wo=Wo, cin=Cin),
        out_shape=jax.ShapeDtypeStruct((N, Ho, Wo, Cmid), _BF),
        grid=(N,),
        in_specs=[
            pl.BlockSpec((1, H + 2, Wo + 1, 2 * Cin), lambda n: (n, 0, 0, 0)),
            pl.BlockSpec((9 * Cin, Cmid), lambda n: (0, 0)),
            pl.BlockSpec((1, Cmid), lambda n: (0, 0)),
            pl.BlockSpec((9, Cmid, Cmid), lambda n: (0, 0, 0)),
            pl.BlockSpec((1, Cmid), lambda n: (0, 0)),
        ],
        out_specs=pl.BlockSpec((1, Ho, Wo, Cmid), lambda n: (n, 0, 0, 0)),
        compiler_params=pltpu.CompilerParams(
            dimension_semantics=("parallel",),
            vmem_limit_bytes=_VMEM,
        ),
    )(hf, w1r, b1r, w2r, b2r)


def _corr_ext(f2, f3, xs, cf):
    H, W, _ = f2.shape
    scale = 1.0 / math.sqrt(cf)
    posx = (lax.broadcasted_iota(jnp.int32, (1, 1, W + 4), 2) - 2
            ).astype(jnp.float32)
    rc = min(H, max(8, (4 << 20) // max(1, W * W * 4)))
    rows = []
    for r0 in range(0, H, rc):
        r = min(rc, H - r0)
        corr = lax.dot_general(
            f2[r0:r0 + r], f3[r0:r0 + r],
            (((2,), (2,)), ((0,), (0,))),
            preferred_element_type=jnp.float32) * scale
        corr_x = jnp.pad(corr, ((0, 0), (0, 0), (4, 4)))
        hat0 = jnp.maximum(
            0.0, 1.0 - jnp.abs(xs[r0:r0 + r, :, None] - posx))
        js = []
        for j in (-2, -1, 0, 1, 2):
            sl = corr_x[:, :, 2 + j:2 + j + W + 4]
            js.append(jnp.sum(sl * hat0, -1, keepdims=True))
        rows.append(jnp.concatenate(js, -1))
    return jnp.concatenate(rows, 0)


def _head(h, wv_ref, b_ref):
    H, W, C = h.shape
    hp = _hpad(h)
    wv = wv_ref[...].astype(jnp.float32)
    rc = min(H, max(8, (4 << 20) // max(1, W * 9 * C * 4)))
    outs = []
    for r0 in range(0, H, rc):
        r = min(rc, H - r0)
        taps = [hp[r0 + dy:r0 + dy + r, dx:dx + W, :]
                for dy in range(3) for dx in range(3)]
        patch = jnp.concatenate(taps, -1).astype(jnp.float32)
        outs.append(jnp.sum(patch * wv, -1))
    return jnp.concatenate(outs, 0) + b_ref[0, 0]


def _dec_kernel(*refs, cf, scale, has_disp):
    if has_disp:
        f2_ref, f3_ref, dp_ref = refs[:3]
        wrefs = refs[3:-1]
    else:
        f2_ref, f3_ref = refs[:2]
        wrefs = refs[2:-1]
    o_ref = refs[-1]
    f2 = f2_ref[0]
    f3 = f3_ref[0]
    ho, w, _ = f2.shape
    wiota = lax.broadcasted_iota(jnp.int32, (ho, w), 1).astype(jnp.float32)
    if has_disp:
        du = jnp.repeat(jnp.repeat(dp_ref[0], 2, 0), 2, 1) * scale
        xs = du + wiota
    else:
        du = None
        xs = wiota
    ext = _corr_ext(f2, f3, xs, cf)
    if has_disp:
        ext = jnp.concatenate([ext, du[:, :, None]], -1)
    h = jnp.concatenate([f2, ext.astype(_BF)], -1)
    for i in range(5):
        h = _conv_s1(h, wrefs[2 * i][...], wrefs[2 * i + 1][...], relu=True)
    o_ref[0] = _head(h, wrefs[10], wrefs[11])


def _dec_level(F, disp_prev, layers, cf, scale):
    N8, Ho, W, _ = F.shape
    N = N8 // 2
    has_disp = disp_prev is not None
    ops = []
    in_specs = [
        pl.BlockSpec((1, Ho, W, cf), lambda n: (n, 0, 0, 0)),
        pl.BlockSpec((1, Ho, W, cf), lambda n: (n + N, 0, 0, 0)),
    ]
    if has_disp:
        Hp, Wp = disp_prev.shape[1:3]
        ops.append(disp_prev)
        in_specs.append(pl.BlockSpec((1, Hp, Wp), lambda n: (n, 0, 0)))
    for i, (wl, bl) in enumerate(layers):
        cin, cout = wl.shape[2], wl.shape[3]
        if i == 5:
            ops.append(wl.reshape(1, 1, 9 * cin).astype(_BF))
            in_specs.append(pl.BlockSpec((1, 1, 9 * cin), lambda n: (0, 0, 0)))
            ops.append(bl.reshape(1, 1).astype(jnp.float32))
            in_specs.append(pl.BlockSpec((1, 1), lambda n: (0, 0)))
        else:
            ops.append(wl.reshape(9, cin, cout).astype(_BF))
            in_specs.append(pl.BlockSpec((9, cin, cout), lambda n: (0, 0, 0)))
            ops.append(bl.reshape(1, cout).astype(jnp.float32))
            in_specs.append(pl.BlockSpec((1, cout), lambda n: (0, 0)))
    return pl.pallas_call(
        functools.partial(_dec_kernel, cf=cf, scale=scale, has_disp=has_disp),
        out_shape=jax.ShapeDtypeStruct((N, Ho, W), jnp.float32),
        grid=(N,),
        in_specs=in_specs,
        out_specs=pl.BlockSpec((1, Ho, W), lambda n: (n, 0, 0)),
        compiler_params=pltpu.CompilerParams(
            dimension_semantics=("parallel",),
            vmem_limit_bytes=_VMEM,
        ),
    )(F, F, *ops)


def kernel(image2, image3, fea_0_c1_w, fea_0_c1_b, fea_0_c2_w, fea_0_c2_b, fea_1_c1_w, fea_1_c1_b, fea_1_c2_w, fea_1_c2_b, fea_2_c1_w, fea_2_c1_b, fea_2_c2_w, fea_2_c2_b, fea_3_c1_w, fea_3_c1_b, fea_3_c2_w, fea_3_c2_b, fea_4_c1_w, fea_4_c1_b, fea_4_c2_w, fea_4_c2_b, fea_5_c1_w, fea_5_c1_b, fea_5_c2_w, fea_5_c2_b, dec_6_0_w, dec_6_0_b, dec_6_1_w, dec_6_1_b, dec_6_2_w, dec_6_2_b, dec_6_3_w, dec_6_3_b, dec_6_4_w, dec_6_4_b, dec_6_5_w, dec_6_5_b, dec_5_0_w, dec_5_0_b, dec_5_1_w, dec_5_1_b, dec_5_2_w, dec_5_2_b, dec_5_3_w, dec_5_3_b, dec_5_4_w, dec_5_4_b, dec_5_5_w, dec_5_5_b, dec_4_0_w, dec_4_0_b, dec_4_1_w, dec_4_1_b, dec_4_2_w, dec_4_2_b, dec_4_3_w, dec_4_3_b, dec_4_4_w, dec_4_4_b, dec_4_5_w, dec_4_5_b, dec_3_0_w, dec_3_0_b, dec_3_1_w, dec_3_1_b, dec_3_2_w, dec_3_2_b, dec_3_3_w, dec_3_3_b, dec_3_4_w, dec_3_4_b, dec_3_5_w, dec_3_5_b, dec_2_0_w, dec_2_0_b, dec_2_1_w, dec_2_1_b, dec_2_2_w, dec_2_2_b, dec_2_3_w, dec_2_3_b, dec_2_4_w, dec_2_4_b, dec_2_5_w, dec_2_5_b):
    args = locals()
    p = [args[n] for n in _ARG_ORDER]
    return _kernel_impl(image2, image3, *p)


_ARG_ORDER = []
for _l in range(6):
    for _c in ("c1", "c2"):
        _ARG_ORDER += [f"fea_{_l}_{_c}_w", f"fea_{_l}_{_c}_b"]
for _l in (6, 5, 4, 3, 2):
    for _i in range(6):
        _ARG_ORDER += [f"dec_{_l}_{_i}_w", f"dec_{_l}_{_i}_b"]


def _kernel_impl(image2, image3, *p):
    names = []
    for lvl in range(6):
        for c in ("c1", "c2"):
            names += [f"fea_{lvl}_{c}_w", f"fea_{lvl}_{c}_b"]
    for lvl in (6, 5, 4, 3, 2):
        for i in range(6):
            names += [f"dec_{lvl}_{i}_w", f"dec_{lvl}_{i}_b"]
    prm = dict(zip(names, p))

    x2 = jnp.transpose(image2, (0, 2, 3, 1)).astype(_BF)
    x3 = jnp.transpose(image3, (0, 2, 3, 1)).astype(_BF)
    h = jnp.concatenate([x2, x3], 0)

    feats = []
    tile_rows = {0: 32, 1: 16}
    for lvl in range(6):
        wb = (prm[f"fea_{lvl}_c1_w"], prm[f"fea_{lvl}_c1_b"],
              prm[f"fea_{lvl}_c2_w"], prm[f"fea_{lvl}_c2_b"])
        if lvl in tile_rows:
            h = _fea_level_tiled(h, *wb, tile_rows[lvl])
        else:
            h = _fea_level(h, *wb)
        feats.append(h)

    cfs = {2: 32, 3: 64, 4: 96, 5: 128, 6: 192}
    disp = None
    disps = {}
    for lvl in (6, 5, 4, 3, 2):
        layers = [(prm[f"dec_{lvl}_{i}_w"], prm[f"dec_{lvl}_{i}_b"])
                  for i in range(6)]
        disp = _dec_level(feats[lvl - 1], disp, layers, cfs[lvl],
                          20.0 / (2 ** lvl))
        disps[lvl] = disp

    out = tuple(disps[lvl][:, None, :, :] for lvl in (2, 3, 4, 5, 6))
    return out
